# speedup vs baseline: 8.7553x; 8.7553x over previous
import functools

import jax
import jax.numpy as jnp
from jax import lax
from jax.experimental import pallas as pl
from jax.experimental.pallas import tpu as pltpu
from jax.experimental.pallas import tpu_sc as plsc

N = 10000
E = 320000
D = 128
ALPHA = 0.5

NC = 2
NS = 16
NW = NC * NS
B = 128
NB = -(-E // (NW * B))
CHUNK = NB * B
E_PAD = NW * CHUNK
ACC_ROWS = 10240
RPS = ACC_ROWS // NS
TRASH = N


def _sc_agg(table, src, dst):
    mesh = plsc.VectorSubcoreMesh(core_axis_name="c", subcore_axis_name="s")

    @functools.partial(
        pl.kernel,
        out_type=jax.ShapeDtypeStruct((NC, ACC_ROWS, D), jnp.float32),
        mesh=mesh,
        scratch_types=[
            pltpu.VMEM((1, B), jnp.int32),
            pltpu.VMEM((1, B), jnp.int32),
            pltpu.VMEM((B, D), jnp.float32),
            pltpu.VMEM((16, D), jnp.float32),
            pltpu.VMEM_SHARED((ACC_ROWS, D), jnp.float32),
            pltpu.SemaphoreType.DMA,
        ],
    )
    def k(table_h, src_h, dst_h, out_h, srcv, dstv, rows, zb, acc, sem):
        c = lax.axis_index("c")
        s = lax.axis_index("s")
        w = c * NS + s
        zv = jnp.zeros((16,), jnp.float32)
        for i in range(16):
            for j in range(D // 16):
                zb[i, pl.ds(j * 16, 16)] = zv
        for t in range(RPS // 16):
            pltpu.sync_copy(zb, acc.at[pl.ds(s * RPS + t * 16, 16)])
        plsc.subcore_barrier()

        def body(i, carry):
            off = w * CHUNK + i * B
            pltpu.sync_copy(src_h.at[pl.ds(off, B)], srcv.at[0])
            pltpu.sync_copy(dst_h.at[pl.ds(off, B)], dstv.at[0])
            pltpu.async_copy(table_h.at[srcv.at[0]], rows, sem).wait()
            pltpu.sync_copy(rows, acc.at[dstv.at[0]], add=True)
            return carry

        lax.fori_loop(0, NB, body, 0)
        plsc.subcore_barrier()
        pltpu.sync_copy(acc.at[pl.ds(s * RPS, RPS)],
                        out_h.at[c, pl.ds(s * RPS, RPS)])

    return k(table, src, dst)


def _sc_degree(dst):
    mesh = plsc.VectorSubcoreMesh(core_axis_name="c", subcore_axis_name="s")

    @functools.partial(
        pl.kernel,
        out_type=jax.ShapeDtypeStruct((NC, ACC_ROWS, 16), jnp.float32),
        mesh=mesh,
        scratch_types=[
            pltpu.VMEM((1, B), jnp.int32),
            pltpu.VMEM((B, 16), jnp.float32),
            pltpu.VMEM((16, 16), jnp.float32),
            pltpu.VMEM_SHARED((ACC_ROWS, 16), jnp.float32),
        ],
    )
    def k(dst_h, out_h, dstv, ones, zb, acc):
        c = lax.axis_index("c")
        s = lax.axis_index("s")
        w = c * NS + s
        ov = jnp.ones((16,), jnp.float32)
        zv = jnp.zeros((16,), jnp.float32)
        for i in range(B):
            ones[i, pl.ds(0, 16)] = ov
        for i in range(16):
            zb[i, pl.ds(0, 16)] = zv
        for t in range(RPS // 16):
            pltpu.sync_copy(zb, acc.at[pl.ds(s * RPS + t * 16, 16)])
        plsc.subcore_barrier()

        def body(i, carry):
            off = w * CHUNK + i * B
            pltpu.sync_copy(dst_h.at[pl.ds(off, B)], dstv.at[0])
            pltpu.sync_copy(ones, acc.at[dstv.at[0]], add=True)
            return carry

        lax.fori_loop(0, NB, body, 0)
        plsc.subcore_barrier()
        pltpu.sync_copy(acc.at[pl.ds(s * RPS, RPS)],
                        out_h.at[c, pl.ds(s * RPS, RPS)])

    return k(dst)


_R = 1000


def _tc_dinv(dparts):
    def body(dp_ref, o_ref):
        dp = dp_ref[...]
        deg = dp[0, :, 0:1] + dp[1, :, 0:1] + 1.0
        o_ref[...] = jnp.broadcast_to(lax.rsqrt(deg), (_R, D))

    return pl.pallas_call(
        body,
        grid=(N // _R,),
        in_specs=[pl.BlockSpec((NC, _R, 16), lambda i: (0, i, 0))],
        out_specs=pl.BlockSpec((_R, D), lambda i: (i, 0)),
        out_shape=jax.ShapeDtypeStruct((N, D), jnp.float32),
    )(dparts)


def _tc_first(x, W, dinvb):
    def body(x_ref, w_ref, dv_ref, o_ref):
        h = lax.dot_general(x_ref[...], w_ref[...], (((1,), (1,)), ((), ())),
                            preferred_element_type=jnp.float32)
        o_ref[...] = dv_ref[...] * h

    return pl.pallas_call(
        body,
        grid=(N // _R,),
        in_specs=[
            pl.BlockSpec((_R, D), lambda i: (i, 0)),
            pl.BlockSpec((D, D), lambda i: (0, 0)),
            pl.BlockSpec((_R, D), lambda i: (i, 0)),
        ],
        out_specs=pl.BlockSpec((_R, D), lambda i: (i, 0)),
        out_shape=jax.ShapeDtypeStruct((N, D), jnp.float32),
    )(x, W, dinvb)


def _tc_mid(p, hp, xres, dinvb, b, g, be, Wn):
    def body(p_ref, hp_ref, xr_ref, dv_ref, b_ref, g_ref, be_ref, wn_ref,
             xn_ref, hn_ref):
        pv = p_ref[...]
        dinv = dv_ref[...]
        conv = dinv * (pv[0] + pv[1] + hp_ref[...]) + b_ref[...]
        mu = jnp.mean(conv, axis=-1, keepdims=True)
        var = jnp.mean((conv - mu) ** 2, axis=-1, keepdims=True)
        ln = (conv - mu) / jnp.sqrt(var + 1e-5) * g_ref[...] + be_ref[...]
        xn = jnp.maximum(ALPHA * ln + (1.0 - ALPHA) * xr_ref[...], 0.0)
        xn_ref[...] = xn
        hw = lax.dot_general(xn, wn_ref[...], (((1,), (1,)), ((), ())),
                             preferred_element_type=jnp.float32)
        hn_ref[...] = dinv * hw

    return pl.pallas_call(
        body,
        grid=(N // _R,),
        in_specs=[
            pl.BlockSpec((NC, _R, D), lambda i: (0, i, 0)),
            pl.BlockSpec((_R, D), lambda i: (i, 0)),
            pl.BlockSpec((_R, D), lambda i: (i, 0)),
            pl.BlockSpec((_R, D), lambda i: (i, 0)),
            pl.BlockSpec((1, D), lambda i: (0, 0)),
            pl.BlockSpec((1, D), lambda i: (0, 0)),
            pl.BlockSpec((1, D), lambda i: (0, 0)),
            pl.BlockSpec((D, D), lambda i: (0, 0)),
        ],
        out_specs=(
            pl.BlockSpec((_R, D), lambda i: (i, 0)),
            pl.BlockSpec((_R, D), lambda i: (i, 0)),
        ),
        out_shape=(
            jax.ShapeDtypeStruct((N, D), jnp.float32),
            jax.ShapeDtypeStruct((N, D), jnp.float32),
        ),
    )(p, hp, xres, dinvb, b, g, be, Wn)


def _tc_last(p, hp, dinvb, b):
    def body(p_ref, hp_ref, dv_ref, b_ref, o_ref):
        pv = p_ref[...]
        o_ref[...] = dv_ref[...] * (pv[0] + pv[1] + hp_ref[...]) + b_ref[...]

    return pl.pallas_call(
        body,
        grid=(N // _R,),
        in_specs=[
            pl.BlockSpec((NC, _R, D), lambda i: (0, i, 0)),
            pl.BlockSpec((_R, D), lambda i: (i, 0)),
            pl.BlockSpec((_R, D), lambda i: (i, 0)),
            pl.BlockSpec((1, D), lambda i: (0, 0)),
        ],
        out_specs=pl.BlockSpec((_R, D), lambda i: (i, 0)),
        out_shape=jax.ShapeDtypeStruct((N, D), jnp.float32),
    )(p, hp, dinvb, b)


def kernel(x, edge_index, W1, b1, g1, be1, W2, b2, g2, be2, W3, b3):
    pad = E_PAD - E
    srcp = jnp.concatenate([edge_index[0], jnp.zeros((pad,), jnp.int32)])
    dstp = jnp.concatenate([edge_index[1], jnp.full((pad,), TRASH, jnp.int32)])

    dparts = _sc_degree(dstp)
    dinvb = _tc_dinv(dparts)

    h1p = _tc_first(x, W1, dinvb)
    p1 = _sc_agg(h1p, srcp, dstp)
    x1, h2p = _tc_mid(p1, h1p, x, dinvb, b1.reshape(1, D), g1.reshape(1, D),
                      be1.reshape(1, D), W2)
    p2 = _sc_agg(h2p, srcp, dstp)
    _, h3p = _tc_mid(p2, h2p, x1, dinvb, b2.reshape(1, D), g2.reshape(1, D),
                     be2.reshape(1, D), W3)
    p3 = _sc_agg(h3p, srcp, dstp)
    return _tc_last(p3, h3p, dinvb, b3.reshape(1, D))
